# baseline (speedup 1.0000x reference)
import functools
import jax
import jax.numpy as jnp
from jax import lax
from jax.experimental import pallas as pl
from jax.experimental.pallas import tpu as pltpu
from jax.experimental.pallas import tpu_sc as plsc

_N = 524288
_NLOW = 262144
_NT = 16
_NE = _N // _NT
_G = _NE // 16
_TOP = -(2 ** 31)


def _ukey(x):
    k = plsc.bitcast(x, jnp.int32)
    m = k >> 31
    return k ^ (m | jnp.int32(_TOP))


def _vsqrt(v):
    i = plsc.bitcast(v, jnp.int32)
    y = plsc.bitcast(jnp.int32(0x5F3759DF) - (i >> 1), jnp.float32)
    for _ in range(4):
        y = y * (1.5 - 0.5 * v * y * y)
    return v * y


def _verf(x):
    sgn = jnp.where(x < 0.0, -1.0, 1.0).astype(jnp.float32)
    a = jnp.abs(x)
    t = 1.0 / (1.0 + 0.3275911 * a)
    poly = t * (0.254829592 + t * (-0.284496736 + t * (1.421413741
           + t * (-1.453152027 + t * 1.061405429))))
    return sgn * (1.0 - poly * jnp.exp(-a * a))


def _zero_hist(hist):
    zeros16 = jnp.zeros((16,), jnp.int32)

    def zz(j, _):
        hist[pl.ds(j * 16, 16)] = zeros16
        return 0

    lax.fori_loop(0, 256, zz, 0)


def _merge_and_pick(sid, hist, hmerged, allh, shared_h, r):
    def mg(g, _):
        acc = hist[pl.ds(g * 16, 16)]
        for l in range(1, 16):
            acc = acc + hist[pl.ds(l * 256 + g * 16, 16)]
        hmerged[pl.ds(g * 16, 16)] = acc
        return 0

    lax.fori_loop(0, 16, mg, 0)
    pltpu.sync_copy(hmerged, shared_h.at[sid])
    plsc.subcore_barrier()
    pltpu.sync_copy(shared_h, allh)

    def dec(g, carry):
        nbkt, below, cumbase = carry
        gcnt = allh[0, pl.ds(g * 16, 16)]
        for tl in range(1, 16):
            gcnt = gcnt + allh[tl, pl.ds(g * 16, 16)]
        cum = plsc.cumsum(gcnt) + cumbase
        mask = cum <= r
        nbkt = nbkt + jnp.sum(mask.astype(jnp.int32))
        below = below + jnp.sum(jnp.where(mask, gcnt, 0))
        cumbase = cumbase + jnp.sum(gcnt)
        return (nbkt, below, cumbase)

    nbkt, below, _ = lax.fori_loop(
        0, 16, dec, (jnp.int32(0), jnp.int32(0), jnp.int32(0)))
    return nbkt, below


def _sc_body(x_hbm, out_hbm, xbuf, cbuf, hist, hmerged, allh, shared_h,
             shared_stats, rowbuf, statsbuf, resbuf):
    sid = lax.axis_index("s")
    lane = lax.iota(jnp.int32, 16)
    lane_base = lane * 256
    ones = jnp.ones((16,), jnp.int32)
    fzero = jnp.zeros((16,), jnp.float32)

    pltpu.sync_copy(x_hbm.at[pl.ds(sid * _NE, _NE)], xbuf)

    _zero_hist(hist)

    def s1(j, carry):
        acc_s, acc_ss = carry
        x = xbuf[pl.ds(j * 16, 16)]
        b1 = lax.shift_right_logical(_ukey(x), 24)
        plsc.addupdate_scatter(hist, [lane_base + b1], ones)
        return (acc_s + x, acc_ss + x * x)

    acc_s, acc_ss = lax.fori_loop(0, _G, s1, (fzero, fzero))

    r = jnp.int32(_NLOW)
    sel1, below1 = _merge_and_pick(sid, hist, hmerged, allh, shared_h, r)
    r = r - below1

    _zero_hist(hist)

    def s2(j, carry):
        off, a_sl, a_ssl = carry
        x = xbuf[pl.ds(j * 16, 16)]
        uk = _ukey(x)
        b1 = lax.shift_right_logical(uk, 24)
        match = b1 == sel1
        lt = b1 < sel1
        a_sl = a_sl + jnp.where(lt, x, 0.0)
        a_ssl = a_ssl + jnp.where(lt, x * x, 0.0)
        b2 = lax.shift_right_logical(uk, 16) & 255
        plsc.addupdate_scatter(hist, [lane_base + b2], ones, mask=match)
        plsc.store_compressed(cbuf.at[pl.ds(off, 16)], x, mask=match)
        off = off + jnp.sum(match.astype(jnp.int32))
        return (off, a_sl, a_ssl)

    l2, a_sl, a_ssl = lax.fori_loop(0, _G, s2, (jnp.int32(0), fzero, fzero))

    sel2, below2 = _merge_and_pick(sid, hist, hmerged, allh, shared_h, r)
    r = r - below2
    prefix16 = sel1 * 256 + sel2

    _zero_hist(hist)
    g3 = (l2 + 15) >> 4

    def s3(j, carry):
        off, a_sl, a_ssl = carry
        x = cbuf[pl.ds(j * 16, 16)]
        uk = _ukey(x)
        valid = (j * 16 + lane) < l2
        b2 = lax.shift_right_logical(uk, 16) & 255
        lt = valid & (b2 < sel2)
        a_sl = a_sl + jnp.where(lt, x, 0.0)
        a_ssl = a_ssl + jnp.where(lt, x * x, 0.0)
        match = valid & (b2 == sel2)
        b3 = lax.shift_right_logical(uk, 8) & 255
        plsc.addupdate_scatter(hist, [lane_base + b3], ones, mask=match)
        plsc.store_compressed(xbuf.at[pl.ds(off, 16)], x, mask=match)
        off = off + jnp.sum(match.astype(jnp.int32))
        return (off, a_sl, a_ssl)

    l3, a_sl, a_ssl = lax.fori_loop(0, g3, s3, (jnp.int32(0), a_sl, a_ssl))

    sel3, below3 = _merge_and_pick(sid, hist, hmerged, allh, shared_h, r)
    r = r - below3

    _zero_hist(hist)
    g4 = (l3 + 15) >> 4

    def s4(j, carry):
        a_sl, a_ssl = carry
        x = xbuf[pl.ds(j * 16, 16)]
        uk = _ukey(x)
        valid = (j * 16 + lane) < l3
        b3 = lax.shift_right_logical(uk, 8) & 255
        lt = valid & (b3 < sel3)
        a_sl = a_sl + jnp.where(lt, x, 0.0)
        a_ssl = a_ssl + jnp.where(lt, x * x, 0.0)
        match = valid & (b3 == sel3)
        b4 = uk & 255
        plsc.addupdate_scatter(hist, [lane_base + b4], ones, mask=match)
        return (a_sl, a_ssl)

    a_sl, a_ssl = lax.fori_loop(0, g4, s4, (a_sl, a_ssl))

    sel4, below4 = _merge_and_pick(sid, hist, hmerged, allh, shared_h, r)
    r = r - below4

    def s5(j, carry):
        a_sl, a_ssl = carry
        x = xbuf[pl.ds(j * 16, 16)]
        uk = _ukey(x)
        valid = (j * 16 + lane) < l3
        b3 = lax.shift_right_logical(uk, 8) & 255
        b4 = uk & 255
        lt = valid & (b3 == sel3) & (b4 < sel4)
        a_sl = a_sl + jnp.where(lt, x, 0.0)
        a_ssl = a_ssl + jnp.where(lt, x * x, 0.0)
        return (a_sl, a_ssl)

    a_sl, a_ssl = lax.fori_loop(0, g4, s5, (a_sl, a_ssl))

    io = lane
    row = (jnp.where(io == 0, jnp.sum(a_sl), 0.0)
           + jnp.where(io == 1, jnp.sum(a_ssl), 0.0)
           + jnp.where(io == 2, jnp.sum(acc_s), 0.0)
           + jnp.where(io == 3, jnp.sum(acc_ss), 0.0)).astype(jnp.float32)
    rowbuf[...] = row
    pltpu.sync_copy(rowbuf, shared_stats.at[sid])
    plsc.subcore_barrier()

    @pl.when(sid == 0)
    def _():
        pltpu.sync_copy(shared_stats, statsbuf)
        tot = statsbuf[0, :]
        for tl in range(1, 16):
            tot = tot + statsbuf[tl, :]

        ku = ((sel1 * 256 + sel2) * 256 + sel3) * 256 + sel4
        kuv = jnp.zeros((16,), jnp.int32) + ku
        kbits = jnp.where(kuv < 0, kuv ^ jnp.int32(_TOP), ~kuv)
        tv = plsc.bitcast(kbits, jnp.float32)

        ones_f = jnp.ones((16,), jnp.float32)
        sum_lt = ones_f * tot[0]
        ss_lt = ones_f * tot[1]
        total_s = ones_f * tot[2]
        total_ss = ones_f * tot[3]

        nlow = jnp.float32(_NLOW)
        nhigh = jnp.float32(_N - _NLOW)
        clt = jnp.int32(_NLOW) - r
        fill = nlow - clt.astype(jnp.float32)
        sum_low = sum_lt + fill * tv
        ss_low = ss_lt + fill * tv * tv
        sum_high = total_s - sum_low
        ss_high = total_ss - ss_low

        mu0 = sum_low / nlow
        mu1 = sum_high / nhigh
        var0 = (ss_low - sum_low * mu0) / (nlow - 1.0)
        var1 = (ss_high - sum_high * mu1) / (nhigh - 1.0)
        v0 = _vsqrt(var0)
        v1 = _vsqrt(var1)

        sq2 = jnp.float32(1.4142135623730951)
        inv_sqrt2pi = jnp.float32(0.3989422804014327)
        sigma0 = _vsqrt(v0)
        sigma1 = _vsqrt(v1)
        z0 = (-1.0 - mu0) / sigma0
        z1 = (1.0 - mu1) / sigma1
        mor0 = jnp.exp(-0.5 * z0 * z0) * inv_sqrt2pi / sigma0
        mor1 = jnp.exp(-0.5 * z1 * z1) * inv_sqrt2pi / sigma1
        res = 0.25 * (mu0 + 1.0) * (1.0 - _verf((-mu0 - 1.0) / (sq2 * sigma0)))
        res = res + 0.5 * v0 * mor0
        m3 = 1.0 - mu1
        res = res + 0.25 * m3 * (1.0 + _verf(m3 / (sq2 * sigma1)))
        res = res + 0.5 * v1 * mor1
        res = res + tv * tv

        resbuf[...] = res.astype(jnp.float32)
        pltpu.sync_copy(resbuf, out_hbm)


@jax.jit
def _run(x):
    mesh = plsc.VectorSubcoreMesh(
        core_axis_name="c", subcore_axis_name="s",
        num_cores=1, num_subcores=_NT)
    f = pl.kernel(
        _sc_body,
        out_type=jax.ShapeDtypeStruct((16,), jnp.float32),
        mesh=mesh,
        compiler_params=pltpu.CompilerParams(needs_layout_passes=False),
        scratch_types=[
            pltpu.VMEM((_NE,), jnp.float32),
            pltpu.VMEM((_NE,), jnp.float32),
            pltpu.VMEM((4096,), jnp.int32),
            pltpu.VMEM((256,), jnp.int32),
            pltpu.VMEM((16, 256), jnp.int32),
            pltpu.VMEM_SHARED((16, 256), jnp.int32),
            pltpu.VMEM_SHARED((16, 16), jnp.float32),
            pltpu.VMEM((16,), jnp.float32),
            pltpu.VMEM((16, 16), jnp.float32),
            pltpu.VMEM((16,), jnp.float32),
        ],
    )
    return f(x)


def kernel(x):
    return _run(x)[0]
